# baseline (speedup 1.0000x reference)
import jax
import jax.numpy as jnp
from jax.experimental import pallas as pl
from jax.experimental.pallas import tpu as pltpu

NB = 10
EPS = 1e-6
LANES = 128
SUB = 8
ROWS_TOTAL = 128 * 128
CHUNK = 8192
TILES = CHUNK // SUB
NCHUNK = ROWS_TOTAL // CHUNK
BATCH = 8


def _dice_kernel(x_ref, t_ref, s_ref, out_ref, facc_ref, bacc_ref):
    b = pl.program_id(0)
    c = pl.program_id(1)
    z = jnp.zeros((SUB, LANES), jnp.float32)
    xt_acc = [z] * NB
    q_acc = [z] * NB
    pbits = jnp.zeros((SUB, LANES), jnp.int32)
    nzbits = jnp.zeros((SUB, LANES), jnp.int32)
    one = jnp.int32(1)
    for r in range(TILES):
        x = x_ref[0, r * SUB:(r + 1) * SUB, :]
        t = t_ref[0, r * SUB:(r + 1) * SUB, :]
        s = s_ref[0, r * SUB:(r + 1) * SUB, :]
        xt = x * t
        q = x + t
        bits = jnp.left_shift(one, s)
        pbits = pbits | bits
        nzbits = nzbits | jnp.where(t != 0.0, bits, 0)
        for l in range(NB):
            m = s == (l + 1)
            xt_acc[l] = xt_acc[l] + jnp.where(m, xt, 0.0)
            q_acc[l] = q_acc[l] + jnp.where(m, q, 0.0)
    facc = jnp.stack(xt_acc + q_acc)
    bacc = jnp.stack([pbits, nzbits])

    @pl.when(c == 0)
    def _init():
        facc_ref[b] = facc
        bacc_ref[b] = bacc

    @pl.when(c != 0)
    def _acc():
        facc_ref[b] = facc_ref[b] + facc
        bacc_ref[b] = bacc_ref[b] | bacc

    @pl.when((b == BATCH - 1) & (c == NCHUNK - 1))
    def _finish():
        sums = jnp.sum(facc_ref[...], axis=(2, 3))
        inter = sums[:, 0:NB]
        qsum = sums[:, NB:2 * NB]

        pb = bacc_ref[:, 0]
        nzb = bacc_ref[:, 1]
        ta_nz_cols = []
        present_cols = []
        for l in range(1, NB + 1):
            nz_l = jnp.sum(jnp.right_shift(nzb, l) & 1, axis=(1, 2))
            p_l = jnp.sum(jnp.right_shift(pb, l) & 1)
            ta_nz_cols.append((nz_l > 0).astype(jnp.float32))
            present_cols.append((p_l > 0).astype(jnp.float32))
        ta_nz = jnp.stack(ta_nz_cols, axis=1)
        present = jnp.stack(present_cols)

        denom = qsum + 2.0 * EPS
        batch_loss = (1.0 - 2.0 * inter / denom) * ta_nz
        valid = jnp.sum(ta_nz, axis=0)
        loss_per_block = jnp.sum(batch_loss, axis=0) / jnp.maximum(valid, 1.0)

        num = jnp.sum(present)
        loss = jnp.sum(loss_per_block * present) / num
        out_ref[0, 0] = loss


def kernel(input, target, block):
    B = input.shape[0]
    x = input.reshape(B, ROWS_TOTAL, LANES)
    t = target.reshape(B, ROWS_TOTAL, LANES)
    s = block.reshape(B, ROWS_TOTAL, LANES)

    in_spec = pl.BlockSpec((1, CHUNK, LANES), lambda b, c: (b, c, 0))
    loss = pl.pallas_call(
        _dice_kernel,
        grid=(B, NCHUNK),
        in_specs=[in_spec, in_spec, in_spec],
        out_specs=pl.BlockSpec(memory_space=pltpu.SMEM),
        out_shape=jax.ShapeDtypeStruct((1, 1), jnp.float32),
        scratch_shapes=[
            pltpu.VMEM((BATCH, 2 * NB, SUB, LANES), jnp.float32),
            pltpu.VMEM((BATCH, 2, SUB, LANES), jnp.int32),
        ],
        compiler_params=pltpu.CompilerParams(
            dimension_semantics=("arbitrary", "arbitrary"),
        ),
    )(x, t, s)

    return (loss[0, 0], 0)


# speedup vs baseline: 1.0092x; 1.0092x over previous
import jax
import jax.numpy as jnp
from jax.experimental import pallas as pl
from jax.experimental.pallas import tpu as pltpu

NB = 10
EPS = 1e-6
LANES = 128
SUB = 8
ROWS_TOTAL = 128 * 128
CHUNK = 4096
TILES = CHUNK // SUB
NCHUNK = ROWS_TOTAL // CHUNK
BATCH = 8


def _dice_kernel(x_ref, t_ref, s_ref, out_ref, facc_ref, bacc_ref):
    b = pl.program_id(0)
    c = pl.program_id(1)
    z = jnp.zeros((SUB, LANES), jnp.float32)
    xt_acc = [z] * NB
    q_acc = [z] * NB
    pbits = jnp.zeros((SUB, LANES), jnp.int32)
    nzbits = jnp.zeros((SUB, LANES), jnp.int32)
    one = jnp.int32(1)
    for r in range(TILES):
        x = x_ref[0, r * SUB:(r + 1) * SUB, :]
        t = t_ref[0, r * SUB:(r + 1) * SUB, :]
        s = s_ref[0, r * SUB:(r + 1) * SUB, :]
        xt = x * t
        q = x + t
        bits = jnp.left_shift(one, s)
        pbits = pbits | bits
        nzbits = nzbits | jnp.where(t != 0.0, bits, 0)
        for l in range(NB):
            m = s == (l + 1)
            xt_acc[l] = xt_acc[l] + jnp.where(m, xt, 0.0)
            q_acc[l] = q_acc[l] + jnp.where(m, q, 0.0)
    facc = jnp.stack(xt_acc + q_acc)
    bacc = jnp.stack([pbits, nzbits])

    @pl.when(c == 0)
    def _init():
        facc_ref[b] = facc
        bacc_ref[b] = bacc

    @pl.when(c != 0)
    def _acc():
        facc_ref[b] = facc_ref[b] + facc
        bacc_ref[b] = bacc_ref[b] | bacc

    @pl.when((b == BATCH - 1) & (c == NCHUNK - 1))
    def _finish():
        sums = jnp.sum(facc_ref[...], axis=(2, 3))
        inter = sums[:, 0:NB]
        qsum = sums[:, NB:2 * NB]

        pb = bacc_ref[:, 0]
        nzb = bacc_ref[:, 1]
        ta_nz_cols = []
        present_cols = []
        for l in range(1, NB + 1):
            nz_l = jnp.sum(jnp.right_shift(nzb, l) & 1, axis=(1, 2))
            p_l = jnp.sum(jnp.right_shift(pb, l) & 1)
            ta_nz_cols.append((nz_l > 0).astype(jnp.float32))
            present_cols.append((p_l > 0).astype(jnp.float32))
        ta_nz = jnp.stack(ta_nz_cols, axis=1)
        present = jnp.stack(present_cols)

        denom = qsum + 2.0 * EPS
        batch_loss = (1.0 - 2.0 * inter / denom) * ta_nz
        valid = jnp.sum(ta_nz, axis=0)
        loss_per_block = jnp.sum(batch_loss, axis=0) / jnp.maximum(valid, 1.0)

        num = jnp.sum(present)
        loss = jnp.sum(loss_per_block * present) / num
        out_ref[0, 0] = loss


def kernel(input, target, block):
    B = input.shape[0]
    x = input.reshape(B, ROWS_TOTAL, LANES)
    t = target.reshape(B, ROWS_TOTAL, LANES)
    s = block.reshape(B, ROWS_TOTAL, LANES)

    in_spec = pl.BlockSpec((1, CHUNK, LANES), lambda b, c: (b, c, 0))
    loss = pl.pallas_call(
        _dice_kernel,
        grid=(B, NCHUNK),
        in_specs=[in_spec, in_spec, in_spec],
        out_specs=pl.BlockSpec(memory_space=pltpu.SMEM),
        out_shape=jax.ShapeDtypeStruct((1, 1), jnp.float32),
        scratch_shapes=[
            pltpu.VMEM((BATCH, 2 * NB, SUB, LANES), jnp.float32),
            pltpu.VMEM((BATCH, 2, SUB, LANES), jnp.int32),
        ],
        compiler_params=pltpu.CompilerParams(
            dimension_semantics=("arbitrary", "arbitrary"),
        ),
    )(x, t, s)

    return (loss[0, 0], 0)
